# baseline (speedup 1.0000x reference)
import jax
import jax.numpy as jnp
from jax.experimental import pallas as pl

_BN = 2000
_EPS = 1e-8


def _body(u_ref, x_ref, o_ref):
    xb = x_ref[...]
    u = u_ref[...]
    num = jnp.sum(xb * u, axis=1)
    sq = jnp.sum(xb * xb, axis=1)
    un = jnp.sqrt(jnp.sum(u * u))
    denom = jnp.maximum(un, _EPS) * jnp.maximum(jnp.sqrt(sq), _EPS)
    o_ref[...] = (num / denom)[None, None, :]


def kernel(x, user_embed):
    N, D = x.shape
    grid = N // _BN
    out = pl.pallas_call(
        _body,
        grid=(grid,),
        in_specs=[
            pl.BlockSpec((1, D), lambda i: (0, 0)),
            pl.BlockSpec((_BN, D), lambda i: (i, 0)),
        ],
        out_specs=pl.BlockSpec((1, 1, _BN), lambda i: (i, 0, 0)),
        out_shape=jax.ShapeDtypeStruct((grid, 1, _BN), jnp.float32),
    )(user_embed, x)
    return out.reshape(N)
